# baseline (speedup 1.0000x reference)
import functools

import jax
import jax.numpy as jnp
from jax import lax
from jax.experimental import pallas as pl
from jax.experimental.pallas import tpu as pltpu
from jax.experimental.pallas import tpu_sc as plsc

_D = 32
_B = 4096 * 200
_SCALE = float(_D) ** 0.5

_info = plsc.get_sparse_core_info()
_NC, _NS, _L = _info.num_cores, _info.num_subcores, _info.num_lanes
_NW = _NC * _NS

_GRP = 128
_GRP_PER_CHUNK = 8
_CHUNK = _GRP * _GRP_PER_CHUNK
_B_PER_W = _B // _NW
_N_CHUNKS = _B_PER_W // _CHUNK
_UNROLL = 8

_mesh = plsc.VectorSubcoreMesh(core_axis_name="c", subcore_axis_name="s")


@functools.partial(
    pl.kernel,
    mesh=_mesh,
    out_type=jax.ShapeDtypeStruct((_B, _D), jnp.float32),
    scratch_types=[
        pltpu.VMEM((2, _GRP_PER_CHUNK, _GRP), jnp.int32),
        pltpu.VMEM((2, _CHUNK, _D), jnp.float32),
        pltpu.SemaphoreType.DMA,
        pltpu.SemaphoreType.DMA,
    ],
    compiler_params=pltpu.CompilerParams(use_tc_tiling_on_sc=False),
)
def _emb_lookup(table_hbm, idx_hbm, out_hbm, idx_v, rows_v, gsem, osem):
    wid = lax.axis_index("s") * _NC + lax.axis_index("c")
    base = wid * _B_PER_W

    def stage(ci, slot):
        off = pl.multiple_of(base + ci * _CHUNK, _CHUNK)
        idx_row = pl.multiple_of(off // _GRP, _GRP_PER_CHUNK)
        pltpu.sync_copy(idx_hbm.at[pl.ds(idx_row, _GRP_PER_CHUNK)],
                        idx_v.at[slot])
        for j in range(_GRP_PER_CHUNK):
            pltpu.async_copy(
                table_hbm.at[idx_v.at[slot, j]],
                rows_v.at[slot, pl.ds(j * _GRP, _GRP)],
                gsem,
            )

    def wait_gathers(slot):
        for j in range(_GRP_PER_CHUNK):
            pltpu.make_async_copy(
                table_hbm.at[idx_v.at[slot, j]],
                rows_v.at[slot, pl.ds(j * _GRP, _GRP)],
                gsem,
            ).wait()

    def out_descriptor(ci, slot):
        off = pl.multiple_of(base + ci * _CHUNK, _CHUNK)
        return pltpu.make_async_copy(rows_v.at[slot],
                                     out_hbm.at[pl.ds(off, _CHUNK)], osem)

    stage(0, 0)

    def chunk_body(ci, carry):
        slot = lax.rem(ci, 2)
        nslot = 1 - slot

        wait_gathers(slot)

        @pl.when(ci >= 1)
        def _():
            out_descriptor(ci - 1, nslot).wait()

        @pl.when(ci + 1 < _N_CHUNKS)
        def _():
            stage(ci + 1, nslot)

        def scale_rows(r, c):
            rbase = r * _UNROLL
            for u in range(_UNROLL):
                for h in range(2):
                    sl = pl.ds(h * _L, _L)
                    rows_v[slot, rbase + u, sl] = (
                        rows_v[slot, rbase + u, sl] * _SCALE)
            return c

        lax.fori_loop(0, _CHUNK // _UNROLL, scale_rows, 0)

        out_descriptor(ci, slot).start()
        return carry

    lax.fori_loop(0, _N_CHUNKS, chunk_body, 0)

    out_descriptor(_N_CHUNKS - 1, lax.rem(_N_CHUNKS - 1, 2)).wait()


def kernel(x, table):
    idx = x.reshape(_B // _GRP, _GRP).astype(jnp.int32)
    out = _emb_lookup(table, idx)
    return out.reshape(x.shape[0], x.shape[1], _D)


# speedup vs baseline: 1.0500x; 1.0500x over previous
import functools

import jax
import jax.numpy as jnp
from jax import lax
from jax.experimental import pallas as pl
from jax.experimental.pallas import tpu as pltpu
from jax.experimental.pallas import tpu_sc as plsc

_D = 32
_B1 = 4096
_B2 = 200
_B = _B1 * _B2
_SCALE = float(_D) ** 0.5

_info = plsc.get_sparse_core_info()
_NC, _NS, _L = _info.num_cores, _info.num_subcores, _info.num_lanes
_NW = _NC * _NS

_GRP = 128
_TOK = 512
_GPB = _TOK // _GRP
_BLK_PER_ROW = _B1 // _TOK
_NBLK = _B // _TOK
_BPW = _NBLK // _NW

_mesh = plsc.VectorSubcoreMesh(core_axis_name="c", subcore_axis_name="s")


@functools.partial(
    pl.kernel,
    mesh=_mesh,
    out_type=jax.ShapeDtypeStruct((_B2, _D, _B1), jnp.float32),
    scratch_types=[
        pltpu.VMEM((2, _GPB, _GRP), jnp.int32),
        pltpu.VMEM((2, _TOK, _D), jnp.float32),
        pltpu.VMEM((2, _D, _TOK), jnp.float32),
        pltpu.SemaphoreType.DMA,
        pltpu.SemaphoreType.DMA,
        pltpu.SemaphoreType.DMA((2,)),
    ],
    compiler_params=pltpu.CompilerParams(use_tc_tiling_on_sc=False,
                                         needs_layout_passes=False),
)
def _emb_lookup(table_hbm, x3_hbm, out_hbm, idx_v, rows_v, tbuf, gsem, isem,
                osem):
    wid = lax.axis_index("s") * _NC + lax.axis_index("c")
    first = wid * _BPW

    def loc(t):
        f = first + t
        return f // _BLK_PER_ROW, lax.rem(f, _BLK_PER_ROW)

    def idx_copy(t, slot):
        b2, bb = loc(t)
        return pltpu.make_async_copy(
            x3_hbm.at[b2, pl.ds(bb * _GPB, _GPB)], idx_v.at[slot], isem)

    def gather_descs(slot):
        return [
            pltpu.make_async_copy(
                table_hbm.at[idx_v.at[slot, j]],
                rows_v.at[slot, pl.ds(j * _GRP, _GRP)],
                gsem,
            )
            for j in range(_GPB)
        ]

    def out_desc(t, slot):
        b2, bb = loc(t)
        return pltpu.make_async_copy(
            tbuf.at[slot], out_hbm.at[b2, :, pl.ds(bb * _TOK, _TOK)],
            osem.at[slot])

    lane = jnp.arange(_L, dtype=jnp.int32)
    halves = ((0, lane), (1, lane + _L))

    def compute(slot):
        svec = jnp.full((_L,), slot, jnp.int32)

        def body(r, c):
            tok = r * 4
            for u in range(4):
                cvec = jnp.full((_L,), tok + u, jnp.int32)
                for h, ridx in halves:
                    v = rows_v[slot, tok + u, pl.ds(h * _L, _L)] * _SCALE
                    plsc.store_scatter(tbuf, [svec, ridx, cvec], v)
            return c

        lax.fori_loop(0, _TOK // 4, body, 0)

    idx_copy(0, 0).start()
    idx_copy(0, 0).wait()
    for d in gather_descs(0):
        d.start()
    idx_copy(1, 1).start()

    def block_body(t, carry):
        slot = lax.rem(t, 2)
        nslot = 1 - slot

        for d in gather_descs(slot):
            d.wait()

        @pl.when(t + 1 < _BPW)
        def _():
            idx_copy(t + 1, nslot).wait()
            for d in gather_descs(nslot):
                d.start()

        @pl.when(t + 2 < _BPW)
        def _():
            idx_copy(t + 2, slot).start()

        @pl.when(t >= 2)
        def _():
            out_desc(t - 2, slot).wait()

        compute(slot)
        out_desc(t, slot).start()
        return carry

    lax.fori_loop(0, _BPW, block_body, 0)

    out_desc(_BPW - 2, lax.rem(_BPW - 2, 2)).wait()
    out_desc(_BPW - 1, lax.rem(_BPW - 1, 2)).wait()


def kernel(x, table):
    x3 = jnp.transpose(x).reshape(_B2, _B1 // _GRP, _GRP).astype(jnp.int32)
    out_t = _emb_lookup(table, x3)
    return jnp.transpose(out_t, (2, 0, 1))
